# Initial kernel scaffold; baseline (speedup 1.0000x reference)
import jax
import jax.numpy as jnp
from jax.experimental import pallas as pl


def kernel(x, table):
    raise NotImplementedError("write your pallas kernel here")


# speedup vs baseline: 1.4154x; 1.4154x over previous
import functools

import jax
import jax.numpy as jnp
from jax import lax
from jax.experimental import pallas as pl
from jax.experimental.pallas import tpu as pltpu
from jax.experimental.pallas import tpu_sc as plsc

D = 32
L = 16
NCORE = 2
NSUB = 16
NW = NCORE * NSUB
KB = 8
GW = 128
CHUNK = KB * GW
B_TOTAL = 4096 * 200
BPW = B_TOTAL // NW
NCHUNK = BPW // CHUNK


@functools.partial(
    pl.kernel,
    mesh=plsc.VectorSubcoreMesh(core_axis_name="c", subcore_axis_name="s"),
    out_type=jax.ShapeDtypeStruct((B_TOTAL, D), jnp.float32),
    compiler_params=pltpu.CompilerParams(
        needs_layout_passes=False, use_tc_tiling_on_sc=False),
    scratch_types=[
        pltpu.VMEM((KB, GW), jnp.int32),
        pltpu.VMEM((KB, GW), jnp.int32),
        pltpu.VMEM((CHUNK, D), jnp.float32),
        pltpu.VMEM((CHUNK, D), jnp.float32),
        pltpu.VMEM((L,), jnp.float32),
        pltpu.SemaphoreType.DMA,
        pltpu.SemaphoreType.DMA,
        pltpu.SemaphoreType.DMA,
        pltpu.SemaphoreType.DMA,
        pltpu.SemaphoreType.DMA,
        pltpu.SemaphoreType.DMA,
    ],
)
def _sc_embedding(x_hbm, table_hbm, out_hbm,
                  idx0, idx1, rows0, rows1, msk,
                  si0, si1, sg0, sg1, ss0, ss1):
    idxb = (idx0, idx1)
    rowsb = (rows0, rows1)
    sidx = (si0, si1)
    sgat = (sg0, sg1)
    ssto = (ss0, ss1)

    wid = lax.axis_index("s") * NCORE + lax.axis_index("c")
    base = wid * BPW

    def idx_copy(g, b):
        return pltpu.make_async_copy(x_hbm.at[wid, g], idxb[b], sidx[b])

    def gather_copy(b, j):
        return pltpu.make_async_copy(
            table_hbm.at[idxb[b].at[j]],
            rowsb[b].at[pl.ds(j * GW, GW)],
            sgat[b])

    def gather_drain(b):
        pltpu.make_async_copy(
            table_hbm.at[pl.ds(0, CHUNK)], rowsb[b], sgat[b]).wait()

    def store_copy(g, b):
        return pltpu.make_async_copy(
            rowsb[b], out_hbm.at[pl.ds(base + g * CHUNK, CHUNK)], ssto[b])

    def fixup(b):
        idxr = idxb[b]
        rows = rowsb[b]

        def body(i, carry):
            j = i // KB
            c = (i % KB) * L
            vi = idxr[j, pl.ds(c, L)]
            msk[...] = jnp.where(vi == 0, 0.0, 1.0)
            row0 = i * L
            for r in range(L):
                mr = plsc.load_gather(msk, [jnp.full((L,), r, jnp.int32)])
                for h in range(D // L):
                    seg = rows[row0 + r, pl.ds(h * L, L)]
                    rows[row0 + r, pl.ds(h * L, L)] = seg * mr
            return carry

        lax.fori_loop(0, CHUNK // L, body, 0)

    def process(g, b):
        ob = 1 - b

        @pl.when(g < NCHUNK)
        def _():
            @pl.when(g + 1 < NCHUNK)
            def _():
                idx_copy(g + 1, ob).wait()

                @pl.when(g >= 1)
                def _():
                    store_copy(g - 1, ob).wait()

                for j in range(KB):
                    gather_copy(ob, j).start()

            gather_drain(b)
            fixup(b)

            @pl.when(g + 2 < NCHUNK)
            def _():
                idx_copy(g + 2, b).start()

            store_copy(g, b).start()

    idx_copy(0, 0).start()
    idx_copy(1, 1).start()
    idx_copy(0, 0).wait()
    for j in range(KB):
        gather_copy(0, j).start()

    def outer(G, carry):
        process(2 * G, 0)
        process(2 * G + 1, 1)
        return carry

    lax.fori_loop(0, (NCHUNK + 1) // 2, outer, 0)

    store_copy(NCHUNK - 2, (NCHUNK - 2) % 2).wait()
    store_copy(NCHUNK - 1, (NCHUNK - 1) % 2).wait()


def kernel(x, table):
    rows, cols = x.shape
    assert rows * cols == B_TOTAL and table.shape == (table.shape[0], D)
    xf = x.reshape(NW, NCHUNK, KB, GW)
    out = _sc_embedding(xf, table)
    return out.reshape(rows, cols, D)
